# baseline (speedup 1.0000x reference)
import functools

import jax
import jax.numpy as jnp
from jax import lax
from jax.experimental import pallas as pl
from jax.experimental.pallas import tpu as pltpu
from jax.experimental.pallas import tpu_sc as plsc

N = 128
L = 16
NG = N // L
INF = float("inf")


def _bf_body(adj_hbm, col0_hbm, dist_hbm, pred_hbm, neg_hbm,
             adj_loc, prev_ref, dist_loc, pred_loc, dt_loc, pt_loc,
             negacc_ref, negcomb_ref, outv_ref, shared_dist, negshared):
    s = lax.axis_index("s")
    active = s < NG
    vbase = s * L

    @pl.when(active)
    def _stage():
        pltpu.sync_copy(adj_hbm.at[:, pl.ds(vbase, L)], adj_loc)
        pltpu.sync_copy(col0_hbm, prev_ref)
        dist_loc[0, :] = prev_ref[pl.ds(vbase, L)]
        pred_loc[0, :] = jnp.zeros((L,), jnp.int32)

    def step(i, carry):
        @pl.when(active)
        def _compute():
            chunks = [prev_ref[pl.ds(16 * c, L)] for c in range(N // L)]
            nch = 4
            per = N // nch
            bests = [jnp.full((L,), INF, jnp.float32) for _ in range(nch)]
            bidxs = [jnp.zeros((L,), jnp.int32) for _ in range(nch)]
            for cc in range(nch):
                for t in range(per):
                    u = per * cc + t
                    val = chunks[u // L][u % L] + adj_loc[u, :]
                    cond = val < bests[cc]
                    bests[cc] = jnp.where(cond, val, bests[cc])
                    bidxs[cc] = jnp.where(cond, u, bidxs[cc])
            best, bidx = bests[0], bidxs[0]
            for cc in range(1, nch):
                cond = bests[cc] < best
                best = jnp.where(cond, bests[cc], best)
                bidx = jnp.where(cond, bidxs[cc], bidx)
            dist_loc[i, :] = best
            pred_loc[i, :] = bidx
            pltpu.sync_copy(dist_loc.at[i],
                            shared_dist.at[i % 2, pl.ds(vbase, L)])

        plsc.subcore_barrier()

        @pl.when(active)
        def _fetch():
            pltpu.sync_copy(shared_dist.at[i % 2], prev_ref)

        return carry

    lax.fori_loop(1, N, step, 0)

    @pl.when(active)
    def _finish():
        lastv = prev_ref[pl.ds(vbase, L)]

        def nstep(u, c):
            acc, uvec = c
            pv = plsc.load_gather(prev_ref, [uvec])
            val = pv + adj_loc[u, :]
            return acc | (val < lastv), uvec + 1

        acc, _ = lax.fori_loop(
            0, N, nstep, (jnp.zeros((L,), jnp.bool_),
                          jnp.zeros((L,), jnp.int32)))
        negacc_ref[...] = acc.astype(jnp.int32)
        pltpu.sync_copy(negacc_ref, negshared.at[s])

        it = lax.iota(jnp.int32, L)
        for c in range(N // L):
            rowidx = it + L * c
            for vl in range(L):
                col = jnp.full((L,), vl, jnp.int32)
                dt_loc[vl, pl.ds(L * c, L)] = plsc.load_gather(
                    dist_loc, [rowidx, col])
                pt_loc[vl, pl.ds(L * c, L)] = plsc.load_gather(
                    pred_loc, [rowidx, col])
        pltpu.sync_copy(dt_loc, dist_hbm.at[pl.ds(vbase, L), :])
        pltpu.sync_copy(pt_loc, pred_hbm.at[pl.ds(vbase, L), :])

    plsc.subcore_barrier()

    @pl.when(s == 0)
    def _combine():
        pltpu.sync_copy(negshared, negcomb_ref)
        flag = jnp.zeros((L,), jnp.int32)
        for g in range(NG):
            flag = flag | negcomb_ref[g, :]
        any_neg = jnp.any(flag != 0)
        outv_ref[...] = jnp.where(any_neg, 1, 0) * jnp.ones((L,), jnp.int32)
        pltpu.sync_copy(outv_ref, neg_hbm)


_bf_call = functools.partial(
    pl.kernel,
    out_type=(
        jax.ShapeDtypeStruct((N, N), jnp.float32),
        jax.ShapeDtypeStruct((N, N), jnp.int32),
        jax.ShapeDtypeStruct((L,), jnp.int32),
    ),
    mesh=plsc.VectorSubcoreMesh(core_axis_name="c", subcore_axis_name="s",
                                num_cores=1),
    compiler_params=pltpu.CompilerParams(use_tc_tiling_on_sc=False,
                                         needs_layout_passes=False),
    scratch_types=(
        pltpu.VMEM((N, L), jnp.float32),
        pltpu.VMEM((N,), jnp.float32),
        pltpu.VMEM((N, L), jnp.float32),
        pltpu.VMEM((N, L), jnp.int32),
        pltpu.VMEM((L, N), jnp.float32),
        pltpu.VMEM((L, N), jnp.int32),
        pltpu.VMEM((L,), jnp.int32),
        pltpu.VMEM((NG, L), jnp.int32),
        pltpu.VMEM((L,), jnp.int32),
        pltpu.VMEM_SHARED((2, N), jnp.float32),
        pltpu.VMEM_SHARED((NG, L), jnp.int32),
    ),
)(_bf_body)


@jax.jit
def kernel(adj_matrix, source_node):
    col0 = jnp.full((N,), INF, jnp.float32).at[source_node].set(0.0)
    dist_nm, pred_nm, negv = _bf_call(adj_matrix.astype(jnp.float32), col0)
    return dist_nm, pred_nm, negv[0] != 0


# speedup vs baseline: 1.0131x; 1.0131x over previous
import functools

import jax
import jax.numpy as jnp
from jax import lax
from jax.experimental import pallas as pl
from jax.experimental.pallas import tpu as pltpu
from jax.experimental.pallas import tpu_sc as plsc

N = 128
L = 16
NG = N // L
INF = float("inf")


def _bf_body(adj_hbm, col0_hbm, dist_hbm, pred_hbm, neg_hbm,
             adj_loc, prev_ref, dist_loc, pred_loc,
             negacc_ref, negcomb_ref, outv_ref, shared_dist, negshared):
    s = lax.axis_index("s")
    active = s < NG
    vbase = s * L

    @pl.when(active)
    def _stage():
        pltpu.sync_copy(adj_hbm.at[:, pl.ds(vbase, L)], adj_loc)
        pltpu.sync_copy(col0_hbm, prev_ref)
        dist_loc[0, :] = prev_ref[pl.ds(vbase, L)]
        pred_loc[0, :] = jnp.zeros((L,), jnp.int32)

    def step(i, carry):
        @pl.when(active)
        def _compute():
            chunks = [prev_ref[pl.ds(16 * c, L)] for c in range(N // L)]
            nch = 8
            per = N // nch
            bests = [jnp.full((L,), INF, jnp.float32) for _ in range(nch)]
            bidxs = [jnp.zeros((L,), jnp.int32) for _ in range(nch)]
            for cc in range(nch):
                for t in range(per):
                    u = per * cc + t
                    val = chunks[u // L][u % L] + adj_loc[u, :]
                    cond = val < bests[cc]
                    bests[cc] = jnp.where(cond, val, bests[cc])
                    bidxs[cc] = jnp.where(cond, u, bidxs[cc])
            best, bidx = bests[0], bidxs[0]
            for cc in range(1, nch):
                cond = bests[cc] < best
                best = jnp.where(cond, bests[cc], best)
                bidx = jnp.where(cond, bidxs[cc], bidx)
            dist_loc[i, :] = best
            pred_loc[i, :] = bidx
            pltpu.sync_copy(dist_loc.at[i],
                            shared_dist.at[i % 2, pl.ds(vbase, L)])

        plsc.subcore_barrier()

        @pl.when(active)
        def _fetch():
            pltpu.sync_copy(shared_dist.at[i % 2], prev_ref)

        return carry

    lax.fori_loop(1, N, step, 0)

    @pl.when(active)
    def _finish():
        lastv = prev_ref[pl.ds(vbase, L)]

        def nstep(u, c):
            acc, uvec = c
            pv = plsc.load_gather(prev_ref, [uvec])
            val = pv + adj_loc[u, :]
            return acc | (val < lastv), uvec + 1

        acc, _ = lax.fori_loop(
            0, N, nstep, (jnp.zeros((L,), jnp.bool_),
                          jnp.zeros((L,), jnp.int32)))
        negacc_ref[...] = acc.astype(jnp.int32)
        pltpu.sync_copy(negacc_ref, negshared.at[s])
        pltpu.sync_copy(dist_loc, dist_hbm.at[:, pl.ds(vbase, L)])
        pltpu.sync_copy(pred_loc, pred_hbm.at[:, pl.ds(vbase, L)])

    plsc.subcore_barrier()

    @pl.when(s == 0)
    def _combine():
        pltpu.sync_copy(negshared, negcomb_ref)
        flag = jnp.zeros((L,), jnp.int32)
        for g in range(NG):
            flag = flag | negcomb_ref[g, :]
        any_neg = jnp.any(flag != 0)
        outv_ref[...] = jnp.where(any_neg, 1, 0) * jnp.ones((L,), jnp.int32)
        pltpu.sync_copy(outv_ref, neg_hbm)


_bf_call = functools.partial(
    pl.kernel,
    out_type=(
        jax.ShapeDtypeStruct((N, N), jnp.float32),
        jax.ShapeDtypeStruct((N, N), jnp.int32),
        jax.ShapeDtypeStruct((L,), jnp.int32),
    ),
    mesh=plsc.VectorSubcoreMesh(core_axis_name="c", subcore_axis_name="s",
                                num_cores=1),
    compiler_params=pltpu.CompilerParams(use_tc_tiling_on_sc=False,
                                         needs_layout_passes=False),
    scratch_types=(
        pltpu.VMEM((N, L), jnp.float32),
        pltpu.VMEM((N,), jnp.float32),
        pltpu.VMEM((N, L), jnp.float32),
        pltpu.VMEM((N, L), jnp.int32),
        pltpu.VMEM((L,), jnp.int32),
        pltpu.VMEM((NG, L), jnp.int32),
        pltpu.VMEM((L,), jnp.int32),
        pltpu.VMEM_SHARED((2, N), jnp.float32),
        pltpu.VMEM_SHARED((NG, L), jnp.int32),
    ),
)(_bf_body)


@jax.jit
def kernel(adj_matrix, source_node):
    col0 = jnp.full((N,), INF, jnp.float32).at[source_node].set(0.0)
    dist_it, pred_it, negv = _bf_call(adj_matrix.astype(jnp.float32), col0)
    return dist_it.T, pred_it.T, negv[0] != 0


# speedup vs baseline: 1.0295x; 1.0162x over previous
import functools

import jax
import jax.numpy as jnp
from jax import lax
from jax.experimental import pallas as pl
from jax.experimental.pallas import tpu as pltpu
from jax.experimental.pallas import tpu_sc as plsc

N = 128
L = 16
NG = N // L
INF = float("inf")


def _bf_body(adj_hbm, col0_hbm, dist_hbm, pred_hbm, neg_hbm,
             adj_loc, prev_ref, newd_ref, dist_loc, pred_loc,
             negacc_ref, negcomb_ref, outv_ref, shared_dist, negshared,
             pub_sem):
    s = lax.axis_index("s")
    active = s < NG
    vbase = s * L

    @pl.when(active)
    def _stage():
        pltpu.sync_copy(adj_hbm.at[:, pl.ds(vbase, L)], adj_loc)
        pltpu.sync_copy(col0_hbm, prev_ref)
        dist_loc[0, :] = prev_ref[pl.ds(vbase, L)]
        pred_loc[0, :] = jnp.zeros((L,), jnp.int32)

    def step(i, carry):
        @pl.when(active)
        def _compute():
            chunks = [prev_ref[pl.ds(16 * c, L)] for c in range(N // L)]
            nch = 4
            per = N // nch
            bests = [jnp.full((L,), INF, jnp.float32) for _ in range(nch)]
            bidxs = [jnp.zeros((L,), jnp.int32) for _ in range(nch)]
            for cc in range(nch):
                for t in range(per):
                    u = per * cc + t
                    val = chunks[u // L][u % L] + adj_loc[u, :]
                    cond = val < bests[cc]
                    bests[cc] = jnp.minimum(val, bests[cc])
                    bidxs[cc] = jnp.where(cond, u, bidxs[cc])
            best, bidx = bests[0], bidxs[0]
            for cc in range(1, nch):
                cond = bests[cc] < best
                best = jnp.minimum(bests[cc], best)
                bidx = jnp.where(cond, bidxs[cc], bidx)
            newd_ref[...] = best
            desc = pltpu.async_copy(
                newd_ref, shared_dist.at[i % 2, pl.ds(vbase, L)], pub_sem)
            dist_loc[i, :] = best
            pred_loc[i, :] = bidx
            desc.wait()

        plsc.subcore_barrier()

        @pl.when(active)
        def _fetch():
            pltpu.sync_copy(shared_dist.at[i % 2], prev_ref)

        return carry

    lax.fori_loop(1, N, step, 0)

    @pl.when(active)
    def _finish():
        lastv = prev_ref[pl.ds(vbase, L)]

        def nstep(u, c):
            acc, uvec = c
            pv = plsc.load_gather(prev_ref, [uvec])
            val = pv + adj_loc[u, :]
            return acc | (val < lastv), uvec + 1

        acc, _ = lax.fori_loop(
            0, N, nstep, (jnp.zeros((L,), jnp.bool_),
                          jnp.zeros((L,), jnp.int32)))
        negacc_ref[...] = acc.astype(jnp.int32)
        pltpu.sync_copy(negacc_ref, negshared.at[s])
        pltpu.sync_copy(dist_loc, dist_hbm.at[:, pl.ds(vbase, L)])
        pltpu.sync_copy(pred_loc, pred_hbm.at[:, pl.ds(vbase, L)])

    plsc.subcore_barrier()

    @pl.when(s == 0)
    def _combine():
        pltpu.sync_copy(negshared, negcomb_ref)
        flag = jnp.zeros((L,), jnp.int32)
        for g in range(NG):
            flag = flag | negcomb_ref[g, :]
        any_neg = jnp.any(flag != 0)
        outv_ref[...] = jnp.where(any_neg, 1, 0) * jnp.ones((L,), jnp.int32)
        pltpu.sync_copy(outv_ref, neg_hbm)


_bf_call = functools.partial(
    pl.kernel,
    out_type=(
        jax.ShapeDtypeStruct((N, N), jnp.float32),
        jax.ShapeDtypeStruct((N, N), jnp.int32),
        jax.ShapeDtypeStruct((L,), jnp.int32),
    ),
    mesh=plsc.VectorSubcoreMesh(core_axis_name="c", subcore_axis_name="s",
                                num_cores=1),
    compiler_params=pltpu.CompilerParams(use_tc_tiling_on_sc=False,
                                         needs_layout_passes=False),
    scratch_types=(
        pltpu.VMEM((N, L), jnp.float32),
        pltpu.VMEM((N,), jnp.float32),
        pltpu.VMEM((L,), jnp.float32),
        pltpu.VMEM((N, L), jnp.float32),
        pltpu.VMEM((N, L), jnp.int32),
        pltpu.VMEM((L,), jnp.int32),
        pltpu.VMEM((NG, L), jnp.int32),
        pltpu.VMEM((L,), jnp.int32),
        pltpu.VMEM_SHARED((2, N), jnp.float32),
        pltpu.VMEM_SHARED((NG, L), jnp.int32),
        pltpu.SemaphoreType.DMA,
    ),
)(_bf_body)


@jax.jit
def kernel(adj_matrix, source_node):
    col0 = jnp.full((N,), INF, jnp.float32).at[source_node].set(0.0)
    dist_it, pred_it, negv = _bf_call(adj_matrix.astype(jnp.float32), col0)
    return dist_it.T, pred_it.T, negv[0] != 0


# speedup vs baseline: 1.0661x; 1.0355x over previous
import functools

import jax
import jax.numpy as jnp
from jax import lax
from jax.experimental import pallas as pl
from jax.experimental.pallas import tpu as pltpu
from jax.experimental.pallas import tpu_sc as plsc

N = 128
L = 16
NG = N // L
INF = float("inf")


def _bf_body(adj_hbm, col0_hbm, dist_hbm, pred_hbm, neg_hbm,
             adj_loc, prev_ref, newd_ref, dist_loc, pred_loc,
             negacc_ref, negcomb_ref, outv_ref, shared_dist, negshared,
             pub_sem, flo_sem, fhi_sem):
    s = lax.axis_index("s")
    active = s < NG
    vbase = s * L

    H = N // 2

    @pl.when(active)
    def _stage():
        pltpu.sync_copy(adj_hbm.at[:, pl.ds(vbase, L)], adj_loc)
        pltpu.sync_copy(col0_hbm, prev_ref)
        dist_loc[0, :] = prev_ref[pl.ds(vbase, L)]
        pred_loc[0, :] = jnp.zeros((L,), jnp.int32)
        pltpu.sync_copy(prev_ref.at[pl.ds(vbase, L)],
                        shared_dist.at[0, pl.ds(vbase, L)])

    plsc.subcore_barrier()

    @pl.when(active)
    def _prime():
        pltpu.async_copy(shared_dist.at[0, pl.ds(0, H)],
                         prev_ref.at[pl.ds(0, H)], flo_sem)
        pltpu.async_copy(shared_dist.at[0, pl.ds(H, H)],
                         prev_ref.at[pl.ds(H, H)], fhi_sem)

    def step(i, carry):
        @pl.when(active)
        def _compute():
            nch = 4
            per = N // nch
            bests = [jnp.full((L,), INF, jnp.float32) for _ in range(nch)]
            bidxs = [jnp.zeros((L,), jnp.int32) for _ in range(nch)]

            pltpu.make_async_copy(shared_dist.at[(i - 1) % 2, pl.ds(0, H)],
                                  prev_ref.at[pl.ds(0, H)], flo_sem).wait()
            chunks_lo = [prev_ref[pl.ds(16 * c, L)] for c in range(H // L)]
            for cc in range(nch // 2):
                for t in range(per):
                    u = per * cc + t
                    val = chunks_lo[u // L][u % L] + adj_loc[u, :]
                    cond = val < bests[cc]
                    bests[cc] = jnp.minimum(val, bests[cc])
                    bidxs[cc] = jnp.where(cond, u, bidxs[cc])

            pltpu.make_async_copy(shared_dist.at[(i - 1) % 2, pl.ds(H, H)],
                                  prev_ref.at[pl.ds(H, H)], fhi_sem).wait()
            chunks_hi = [prev_ref[pl.ds(H + 16 * c, L)] for c in range(H // L)]
            for cc in range(nch // 2, nch):
                for t in range(per):
                    u = per * cc + t
                    val = chunks_hi[(u - H) // L][u % L] + adj_loc[u, :]
                    cond = val < bests[cc]
                    bests[cc] = jnp.minimum(val, bests[cc])
                    bidxs[cc] = jnp.where(cond, u, bidxs[cc])

            best, bidx = bests[0], bidxs[0]
            for cc in range(1, nch):
                cond = bests[cc] < best
                best = jnp.minimum(bests[cc], best)
                bidx = jnp.where(cond, bidxs[cc], bidx)
            newd_ref[...] = best
            desc = pltpu.async_copy(
                newd_ref, shared_dist.at[i % 2, pl.ds(vbase, L)], pub_sem)
            dist_loc[i, :] = best
            pred_loc[i, :] = bidx
            desc.wait()

        plsc.subcore_barrier()

        @pl.when(active)
        def _issue_fetch():
            pltpu.async_copy(shared_dist.at[i % 2, pl.ds(0, H)],
                             prev_ref.at[pl.ds(0, H)], flo_sem)
            pltpu.async_copy(shared_dist.at[i % 2, pl.ds(H, H)],
                             prev_ref.at[pl.ds(H, H)], fhi_sem)

        return carry

    lax.fori_loop(1, N, step, 0)

    @pl.when(active)
    def _drain():
        pltpu.make_async_copy(shared_dist.at[(N - 1) % 2, pl.ds(0, H)],
                              prev_ref.at[pl.ds(0, H)], flo_sem).wait()
        pltpu.make_async_copy(shared_dist.at[(N - 1) % 2, pl.ds(H, H)],
                              prev_ref.at[pl.ds(H, H)], fhi_sem).wait()

    @pl.when(active)
    def _finish():
        lastv = prev_ref[pl.ds(vbase, L)]

        def nstep(u, c):
            acc, uvec = c
            pv = plsc.load_gather(prev_ref, [uvec])
            val = pv + adj_loc[u, :]
            return acc | (val < lastv), uvec + 1

        acc, _ = lax.fori_loop(
            0, N, nstep, (jnp.zeros((L,), jnp.bool_),
                          jnp.zeros((L,), jnp.int32)))
        negacc_ref[...] = acc.astype(jnp.int32)
        pltpu.sync_copy(negacc_ref, negshared.at[s])
        pltpu.sync_copy(dist_loc, dist_hbm.at[:, pl.ds(vbase, L)])
        pltpu.sync_copy(pred_loc, pred_hbm.at[:, pl.ds(vbase, L)])

    plsc.subcore_barrier()

    @pl.when(s == 0)
    def _combine():
        pltpu.sync_copy(negshared, negcomb_ref)
        flag = jnp.zeros((L,), jnp.int32)
        for g in range(NG):
            flag = flag | negcomb_ref[g, :]
        any_neg = jnp.any(flag != 0)
        outv_ref[...] = jnp.where(any_neg, 1, 0) * jnp.ones((L,), jnp.int32)
        pltpu.sync_copy(outv_ref, neg_hbm)


_bf_call = functools.partial(
    pl.kernel,
    out_type=(
        jax.ShapeDtypeStruct((N, N), jnp.float32),
        jax.ShapeDtypeStruct((N, N), jnp.int32),
        jax.ShapeDtypeStruct((L,), jnp.int32),
    ),
    mesh=plsc.VectorSubcoreMesh(core_axis_name="c", subcore_axis_name="s",
                                num_cores=1),
    compiler_params=pltpu.CompilerParams(use_tc_tiling_on_sc=False,
                                         needs_layout_passes=False),
    scratch_types=(
        pltpu.VMEM((N, L), jnp.float32),
        pltpu.VMEM((N,), jnp.float32),
        pltpu.VMEM((L,), jnp.float32),
        pltpu.VMEM((N, L), jnp.float32),
        pltpu.VMEM((N, L), jnp.int32),
        pltpu.VMEM((L,), jnp.int32),
        pltpu.VMEM((NG, L), jnp.int32),
        pltpu.VMEM((L,), jnp.int32),
        pltpu.VMEM_SHARED((2, N), jnp.float32),
        pltpu.VMEM_SHARED((NG, L), jnp.int32),
        pltpu.SemaphoreType.DMA,
        pltpu.SemaphoreType.DMA,
        pltpu.SemaphoreType.DMA,
    ),
)(_bf_body)


@jax.jit
def kernel(adj_matrix, source_node):
    col0 = jnp.full((N,), INF, jnp.float32).at[source_node].set(0.0)
    dist_it, pred_it, negv = _bf_call(adj_matrix.astype(jnp.float32), col0)
    return dist_it.T, pred_it.T, negv[0] != 0
